# baseline (speedup 1.0000x reference)
import functools

import jax
import jax.numpy as jnp
from jax import lax
from jax.experimental import pallas as pl
from jax.experimental.pallas import tpu as pltpu
from jax.experimental.pallas import tpu_sc as plsc

NC = 2
NS = 16
C = 128
DH = 64
NB = 4


def _make_sc_agg(N, D, NCHUNK, NP):
    mesh = plsc.VectorSubcoreMesh(core_axis_name="c", subcore_axis_name="s")
    HALF = NCHUNK // 2

    @functools.partial(
        pl.kernel,
        out_type=(
            jax.ShapeDtypeStruct((NC, NP, DH), jnp.float32),
            jax.ShapeDtypeStruct((NC, NP, 16), jnp.float32),
        ),
        mesh=mesh,
        compiler_params=pltpu.CompilerParams(use_tc_tiling_on_sc=False),
        scratch_types=[
            pltpu.VMEM((NCHUNK, C), jnp.int32),
            pltpu.VMEM((NCHUNK, C), jnp.int32),
            [pltpu.VMEM((C, DH), jnp.float32)] * NB,
            pltpu.VMEM((C, 16), jnp.float32),
            pltpu.VMEM_SHARED((NP, DH), jnp.float32),
            pltpu.VMEM_SHARED((NP, 16), jnp.float32),
            [pltpu.SemaphoreType.DMA] * NB,
            pltpu.SemaphoreType.DMA,
        ],
    )
    def sc_agg(xh_hbm, src_hbm, dst_hbm, zrow_hbm, zcnt_hbm, ones_hbm,
               outS_hbm, outC_hbm,
               src_v, dst_v, rows, ones_v, acc_sh, cnt_sh,
               sems, semz):
        c = lax.axis_index("c")
        s = lax.axis_index("s")

        zr = NP // NS
        z0 = s * zr
        dz0 = pltpu.async_copy(zrow_hbm.at[pl.ds(z0, zr)],
                               acc_sh.at[pl.ds(z0, zr)], semz)
        dz1 = pltpu.async_copy(zcnt_hbm.at[pl.ds(z0, zr)],
                               cnt_sh.at[pl.ds(z0, zr)], semz)
        pltpu.sync_copy(src_hbm.at[s], src_v)
        pltpu.sync_copy(dst_hbm.at[s], dst_v)
        pltpu.sync_copy(ones_hbm, ones_v)

        def gather_start(j, b):
            return pltpu.async_copy(xh_hbm.at[c].at[src_v.at[j]], rows[b],
                                    sems[b])

        def gather_wait(j, b):
            pltpu.make_async_copy(xh_hbm.at[c].at[src_v.at[j]], rows[b],
                                  sems[b]).wait()

        for b in range(NB):
            gather_start(b, b)
        dz0.wait()
        dz1.wait()
        plsc.subcore_barrier()

        def ring(i, carry):
            j0 = i * NB
            for b in range(NB):
                j = j0 + b
                gather_wait(j, b)
                pltpu.sync_copy(rows[b], acc_sh.at[dst_v.at[j]], add=True)

                @pl.when(jnp.where(c == 0, j < HALF, j >= HALF))
                def _():
                    pltpu.sync_copy(ones_v, cnt_sh.at[dst_v.at[j]], add=True)

                @pl.when(j + NB < NCHUNK)
                def _():
                    gather_start(j + NB, b)

            return carry

        lax.fori_loop(0, NCHUNK // NB, ring, 0)
        plsc.subcore_barrier()

        rr = NP // NS
        r0 = s * rr
        do0 = pltpu.async_copy(acc_sh.at[pl.ds(r0, rr)],
                               outS_hbm.at[c, pl.ds(r0, rr)], semz)
        do1 = pltpu.async_copy(cnt_sh.at[pl.ds(r0, rr)],
                               outC_hbm.at[c, pl.ds(r0, rr)], semz)
        do0.wait()
        do1.wait()

    return sc_agg


def _dense_body(Sp_ref, Cp_ref, x_ref, W_ref, root_ref, b_ref, out_ref):
    S = jnp.concatenate([Sp_ref[0], Sp_ref[1]], axis=1)
    cnt = Cp_ref[0, :, 0:1] + Cp_ref[1, :, 0:1]
    mean = S * (1.0 / jnp.maximum(cnt, 1.0))
    out_ref[...] = (
        jnp.dot(mean, W_ref[...], preferred_element_type=jnp.float32)
        + jnp.dot(x_ref[...], root_ref[...], preferred_element_type=jnp.float32)
        + b_ref[...]
    )


def _dense(Sp, Cp, x, W, root, b):
    N, D = x.shape
    BN = 2000
    grid = (N // BN,)
    return pl.pallas_call(
        _dense_body,
        grid=grid,
        in_specs=[
            pl.BlockSpec((NC, BN, DH), lambda i: (0, i, 0)),
            pl.BlockSpec((NC, BN, 16), lambda i: (0, i, 0)),
            pl.BlockSpec((BN, D), lambda i: (i, 0)),
            pl.BlockSpec((D, D), lambda i: (0, 0)),
            pl.BlockSpec((D, D), lambda i: (0, 0)),
            pl.BlockSpec((1, D), lambda i: (0, 0)),
        ],
        out_specs=pl.BlockSpec((BN, D), lambda i: (i, 0)),
        out_shape=jax.ShapeDtypeStruct((N, D), jnp.float32),
    )(Sp, Cp, x, W, root, b)


def _prep_edges(edge_index, N, NCHUNK):
    EP = NS * NCHUNK * C
    E = edge_index.shape[1]
    pad = EP - E
    src = jnp.concatenate([edge_index[0], jnp.zeros((pad,), jnp.int32)])
    dst = jnp.concatenate([edge_index[1], jnp.full((pad,), N, jnp.int32)])
    return src.reshape(NS, NCHUNK, C), dst.reshape(NS, NCHUNK, C)


def kernel(x, edge_index_node_rel0_node, edge_index_node_rel1_node,
           W0, root0, b0, W1, root1, b1):
    N, D = x.shape
    E = edge_index_node_rel0_node.shape[1]
    NCHUNK = -(-(-(-E // (NS * C))) // NB) * NB
    NP = -(-(N + 1) // 128) * 128

    sc_agg = _make_sc_agg(N, D, NCHUNK, NP)
    zrow = jnp.zeros((NP, DH), jnp.float32)
    zcnt = jnp.zeros((NP, 16), jnp.float32)
    ones = jnp.ones((C, 16), jnp.float32)

    src0, dst0 = _prep_edges(edge_index_node_rel0_node, N, NCHUNK)
    src1, dst1 = _prep_edges(edge_index_node_rel1_node, N, NCHUNK)

    xh = jnp.stack([x[:, :DH], x[:, DH:]])
    S0, C0 = sc_agg(xh, src0, dst0, zrow, zcnt, ones)
    x1 = _dense(S0, C0, x, W0[0], root0, b0.reshape(1, D))
    x1h = jnp.stack([x1[:, :DH], x1[:, DH:]])
    S1, C1 = sc_agg(x1h, src1, dst1, zrow, zcnt, ones)
    return _dense(S1, C1, x1, W1[0], root1, b1.reshape(1, D))


# speedup vs baseline: 1.3305x; 1.3305x over previous
import functools

import jax
import jax.numpy as jnp
from jax import lax
from jax.experimental import pallas as pl
from jax.experimental.pallas import tpu as pltpu
from jax.experimental.pallas import tpu_sc as plsc

NC = 2
NS = 16
C = 128
DH = 64
NB = 2


def _make_sc_agg(N, D, NCHUNK, NP):
    mesh = plsc.VectorSubcoreMesh(core_axis_name="c", subcore_axis_name="s")
    HALF = NCHUNK // 2

    @functools.partial(
        pl.kernel,
        out_type=(
            jax.ShapeDtypeStruct((NC, NP, DH), jnp.float32),
            jax.ShapeDtypeStruct((NC, NP, 16), jnp.float32),
        ),
        mesh=mesh,
        compiler_params=pltpu.CompilerParams(use_tc_tiling_on_sc=False),
        scratch_types=[
            pltpu.VMEM((NCHUNK, C), jnp.int32),
            pltpu.VMEM((NCHUNK, C), jnp.int32),
            [pltpu.VMEM((C, DH), jnp.float32)] * NB,
            pltpu.VMEM((C, 16), jnp.float32),
            pltpu.VMEM_SHARED((NP, DH), jnp.float32),
            pltpu.VMEM_SHARED((NP, 16), jnp.float32),
            [pltpu.SemaphoreType.DMA] * NB,
            pltpu.SemaphoreType.DMA,
        ],
    )
    def sc_agg(xh_hbm, src_hbm, dst_hbm, zrow_hbm, zcnt_hbm, ones_hbm,
               outS_hbm, outC_hbm,
               src_v, dst_v, rows, ones_v, acc_sh, cnt_sh,
               sems, semz):
        c = lax.axis_index("c")
        s = lax.axis_index("s")

        zr = NP // NS
        z0 = s * zr
        dz0 = pltpu.async_copy(zrow_hbm.at[pl.ds(z0, zr)],
                               acc_sh.at[pl.ds(z0, zr)], semz)
        dz1 = pltpu.async_copy(zcnt_hbm.at[pl.ds(z0, zr)],
                               cnt_sh.at[pl.ds(z0, zr)], semz)
        pltpu.sync_copy(src_hbm.at[s], src_v)
        pltpu.sync_copy(dst_hbm.at[s], dst_v)
        pltpu.sync_copy(ones_hbm, ones_v)

        def gather_start(j, b):
            return pltpu.async_copy(xh_hbm.at[c].at[src_v.at[j]], rows[b],
                                    sems[b])

        def gather_wait(j, b):
            pltpu.make_async_copy(xh_hbm.at[c].at[src_v.at[j]], rows[b],
                                  sems[b]).wait()

        for b in range(NB):
            gather_start(b, b)
        dz0.wait()
        dz1.wait()
        plsc.subcore_barrier()

        def ring(i, carry):
            j0 = i * NB
            for b in range(NB):
                j = j0 + b
                gather_wait(j, b)
                pltpu.sync_copy(rows[b], acc_sh.at[dst_v.at[j]], add=True)

                @pl.when(jnp.where(c == 0, j < HALF, j >= HALF))
                def _():
                    pltpu.sync_copy(ones_v, cnt_sh.at[dst_v.at[j]], add=True)

                @pl.when(j + NB < NCHUNK)
                def _():
                    gather_start(j + NB, b)

            return carry

        lax.fori_loop(0, NCHUNK // NB, ring, 0)
        plsc.subcore_barrier()

        rr = NP // NS
        r0 = s * rr
        do0 = pltpu.async_copy(acc_sh.at[pl.ds(r0, rr)],
                               outS_hbm.at[c, pl.ds(r0, rr)], semz)
        do1 = pltpu.async_copy(cnt_sh.at[pl.ds(r0, rr)],
                               outC_hbm.at[c, pl.ds(r0, rr)], semz)
        do0.wait()
        do1.wait()

    return sc_agg


def _dense_body(Sp_ref, Cp_ref, x_ref, W_ref, root_ref, b_ref, out_ref):
    S = jnp.concatenate([Sp_ref[0], Sp_ref[1]], axis=1)
    cnt = Cp_ref[0, :, 0:1] + Cp_ref[1, :, 0:1]
    mean = S * (1.0 / jnp.maximum(cnt, 1.0))
    out_ref[...] = (
        jnp.dot(mean, W_ref[...], preferred_element_type=jnp.float32)
        + jnp.dot(x_ref[...], root_ref[...], preferred_element_type=jnp.float32)
        + b_ref[...]
    )


def _dense(Sp, Cp, x, W, root, b):
    N, D = x.shape
    BN = 2000
    grid = (N // BN,)
    return pl.pallas_call(
        _dense_body,
        grid=grid,
        in_specs=[
            pl.BlockSpec((NC, BN, DH), lambda i: (0, i, 0)),
            pl.BlockSpec((NC, BN, 16), lambda i: (0, i, 0)),
            pl.BlockSpec((BN, D), lambda i: (i, 0)),
            pl.BlockSpec((D, D), lambda i: (0, 0)),
            pl.BlockSpec((D, D), lambda i: (0, 0)),
            pl.BlockSpec((1, D), lambda i: (0, 0)),
        ],
        out_specs=pl.BlockSpec((BN, D), lambda i: (i, 0)),
        out_shape=jax.ShapeDtypeStruct((N, D), jnp.float32),
    )(Sp, Cp, x, W, root, b)


def _prep_edges(edge_index, N, NCHUNK):
    EP = NS * NCHUNK * C
    E = edge_index.shape[1]
    pad = EP - E
    src = jnp.concatenate([edge_index[0], jnp.zeros((pad,), jnp.int32)])
    dst = jnp.concatenate([edge_index[1], jnp.full((pad,), N, jnp.int32)])
    return src.reshape(NS, NCHUNK, C), dst.reshape(NS, NCHUNK, C)


def kernel(x, edge_index_node_rel0_node, edge_index_node_rel1_node,
           W0, root0, b0, W1, root1, b1):
    N, D = x.shape
    E = edge_index_node_rel0_node.shape[1]
    NCHUNK = -(-(-(-E // (NS * C))) // NB) * NB
    NP = -(-(N + 1) // 128) * 128

    sc_agg = _make_sc_agg(N, D, NCHUNK, NP)
    zrow = jnp.zeros((NP, DH), jnp.float32)
    zcnt = jnp.zeros((NP, 16), jnp.float32)
    ones = jnp.ones((C, 16), jnp.float32)

    src0, dst0 = _prep_edges(edge_index_node_rel0_node, N, NCHUNK)
    src1, dst1 = _prep_edges(edge_index_node_rel1_node, N, NCHUNK)

    xh = jnp.stack([x[:, :DH], x[:, DH:]])
    S0, C0 = sc_agg(xh, src0, dst0, zrow, zcnt, ones)
    x1 = _dense(S0, C0, x, W0[0], root0, b0.reshape(1, D))
    x1h = jnp.stack([x1[:, :DH], x1[:, DH:]])
    S1, C1 = sc_agg(x1h, src1, dst1, zrow, zcnt, ones)
    return _dense(S1, C1, x1, W1[0], root1, b1.reshape(1, D))
